# baseline (speedup 1.0000x reference)
import jax
import jax.numpy as jnp
from jax.experimental import pallas as pl
from jax.experimental.pallas import tpu as pltpu

N = 8192
BM = 1024
BK = 1024


def _proj_kernel(x_ref, w_ref, o_ref):
    o_ref[...] = jnp.dot(x_ref[...].astype(jnp.bfloat16),
                         w_ref[...].astype(jnp.bfloat16),
                         preferred_element_type=jnp.float32)


def _pass1_kernel(a1_ref, p1_ref, a2_ref, p2_ref, s1_ref, b1_ref, w2_ref,
                  o_ref, acc_ref):
    k = pl.program_id(1)
    nk = pl.num_programs(1)

    @pl.when(k == 0)
    def _init():
        acc_ref[...] = jnp.zeros_like(acc_ref)

    sb = s1_ref[...].astype(jnp.bfloat16)
    for i, aref in enumerate((a1_ref, p1_ref, a2_ref, p2_ref)):
        ab = aref[...].astype(jnp.bfloat16)
        acc_ref[:, 32 * i:32 * (i + 1)] += jnp.dot(
            ab, sb[:, 32 * i:32 * (i + 1)], preferred_element_type=jnp.float32)

    @pl.when(k == nk - 1)
    def _fin():
        h = jnp.tanh(acc_ref[...] + b1_ref[0:1, :])
        o_ref[...] = jnp.dot(h.astype(jnp.bfloat16),
                             w2_ref[...].astype(jnp.bfloat16),
                             preferred_element_type=jnp.float32)


def _pass2_kernel(a1_ref, p1_ref, a2_ref, p2_ref, s2_ref, b2_ref, wl_ref,
                  bl_ref, wg_ref, bf_ref,
                  lsm_f_ref, lsm_p1_ref, lsm_p2_ref, fused_ref, acc_ref):
    k = pl.program_id(1)
    nk = pl.num_programs(1)

    @pl.when(k == 0)
    def _init():
        acc_ref[...] = jnp.zeros_like(acc_ref)

    sb = s2_ref[...].astype(jnp.bfloat16)
    for i, aref in enumerate((a1_ref, p1_ref, a2_ref, p2_ref)):
        ab = aref[...].astype(jnp.bfloat16)
        acc_ref[:, 16 * i:16 * (i + 1)] += jnp.dot(
            ab, sb[:, 16 * i:16 * (i + 1)], preferred_element_type=jnp.float32)

    @pl.when(k == nk - 1)
    def _fin():
        h2 = jnp.tanh(acc_ref[...] + b2_ref[0:1, :])
        ob = jnp.dot(h2.astype(jnp.bfloat16), wl_ref[...].astype(jnp.bfloat16),
                     preferred_element_type=jnp.float32) + bl_ref[0:1, :]
        fused = jnp.dot(ob.astype(jnp.bfloat16), wg_ref[...].astype(jnp.bfloat16),
                        preferred_element_type=jnp.float32) + bf_ref[0:1, :]

        def lsm(z):
            m = jnp.max(z, axis=1, keepdims=True)
            e = z - m
            return e - jnp.log(jnp.sum(jnp.exp(e), axis=1, keepdims=True))

        lsm_f_ref[...] = lsm(fused)
        lsm_p1_ref[...] = lsm(ob[:, 8:16])
        lsm_p2_ref[...] = lsm(ob[:, 24:32])
        fused_ref[...] = fused


def kernel(x, A1, P1, A2, P2,
           W1_A1, b1_A1, W2_A1, b2_A1, Wl_A1, bl_A1,
           W1_A2, b1_A2, W2_A2, b2_A2, Wl_A2, bl_A2,
           W1_P1, b1_P1, W2_P1, b2_P1, Wl_P1, bl_P1,
           W1_P2, b1_P2, W2_P2, b2_P2, Wl_P2, bl_P2,
           Wf, bf):
    f32 = jnp.float32
    W1c = jnp.concatenate([W1_A1, W1_P1, W1_A2, W1_P2], axis=1)
    b1c = jnp.broadcast_to(
        jnp.concatenate([b1_A1, b1_P1, b1_A2, b1_P2])[None, :], (8, 128))
    W2bd = jax.scipy.linalg.block_diag(W2_A1, W2_P1, W2_A2, W2_P2)
    b2c = jnp.broadcast_to(
        jnp.concatenate([b2_A1, b2_P1, b2_A2, b2_P2])[None, :], (8, 64))
    Wlbd = jax.scipy.linalg.block_diag(Wl_A1, Wl_P1, Wl_A2, Wl_P2)
    blc = jnp.broadcast_to(
        jnp.concatenate([bl_A1, bl_P1, bl_A2, bl_P2])[None, :], (8, 32))
    Wg = jnp.zeros((32, 8), f32).at[0:8].set(Wf[0:8]).at[16:24].set(Wf[8:16])
    bfc = jnp.broadcast_to(bf[None, :], (8, 8))

    S1 = pl.pallas_call(
        _proj_kernel,
        grid=(N // BM,),
        in_specs=[pl.BlockSpec((BM, 128), lambda i: (i, 0)),
                  pl.BlockSpec((128, 128), lambda i: (0, 0))],
        out_specs=pl.BlockSpec((BM, 128), lambda i: (i, 0)),
        out_shape=jax.ShapeDtypeStruct((N, 128), f32),
    )(x, W1c)

    grid = (N // BM, N // BK)
    a_spec = pl.BlockSpec((BM, BK), lambda i, j: (i, j))

    S2 = pl.pallas_call(
        _pass1_kernel,
        grid=grid,
        in_specs=[a_spec, a_spec, a_spec, a_spec,
                  pl.BlockSpec((BK, 128), lambda i, j: (j, 0)),
                  pl.BlockSpec((8, 128), lambda i, j: (0, 0)),
                  pl.BlockSpec((128, 64), lambda i, j: (0, 0))],
        out_specs=pl.BlockSpec((BM, 64), lambda i, j: (i, 0)),
        out_shape=jax.ShapeDtypeStruct((N, 64), f32),
        scratch_shapes=[pltpu.VMEM((BM, 128), f32)],
        compiler_params=pltpu.CompilerParams(
            dimension_semantics=("parallel", "arbitrary")),
    )(A1, P1, A2, P2, S1, b1c, W2bd)

    out_shapes = [jax.ShapeDtypeStruct((N, 8), f32) for _ in range(4)]
    o_spec = pl.BlockSpec((BM, 8), lambda i, j: (i, 0))
    lsm_f, lsm_p1, lsm_p2, fused = pl.pallas_call(
        _pass2_kernel,
        grid=grid,
        in_specs=[a_spec, a_spec, a_spec, a_spec,
                  pl.BlockSpec((BK, 64), lambda i, j: (j, 0)),
                  pl.BlockSpec((8, 64), lambda i, j: (0, 0)),
                  pl.BlockSpec((64, 32), lambda i, j: (0, 0)),
                  pl.BlockSpec((8, 32), lambda i, j: (0, 0)),
                  pl.BlockSpec((32, 8), lambda i, j: (0, 0)),
                  pl.BlockSpec((8, 8), lambda i, j: (0, 0))],
        out_specs=[o_spec, o_spec, o_spec, o_spec],
        out_shape=out_shapes,
        scratch_shapes=[pltpu.VMEM((BM, 64), f32)],
        compiler_params=pltpu.CompilerParams(
            dimension_semantics=("parallel", "arbitrary")),
    )(A1, P1, A2, P2, S2, b2c, Wlbd, blc, Wg, bfc)

    return (lsm_f, lsm_p1, lsm_p2, fused)


# speedup vs baseline: 1.0408x; 1.0408x over previous
import jax
import jax.numpy as jnp
from jax.experimental import pallas as pl
from jax.experimental.pallas import tpu as pltpu

N = 8192
BM = 256
BK = 4096


def _mega_kernel(x_ref, a1_ref, p1_ref, a2_ref, p2_ref,
                 w1_ref, b1_ref, w2_ref, b2_ref, wl_ref, bl_ref, wg_ref, bf_ref,
                 lsm_f_ref, lsm_p1_ref, lsm_p2_ref, fused_ref,
                 s1_ref, s2_ref, acc_ref):
    ph = pl.program_id(0)
    i = pl.program_id(1)
    j = pl.program_id(2)
    nk = pl.num_programs(2)

    @pl.when((ph == 0) & (i == 0))
    def _build_s1():
        xb = x_ref[pl.ds(j * BK, BK), :].astype(jnp.bfloat16)
        s1_ref[pl.ds(j * BK, BK), :] = jnp.dot(
            xb, w1_ref[...].astype(jnp.bfloat16),
            preferred_element_type=jnp.float32)

    @pl.when(j == 0)
    def _init():
        acc_ref[...] = jnp.zeros_like(acc_ref)

    arefs = (a1_ref, p1_ref, a2_ref, p2_ref)

    @pl.when(ph == 0)
    def _layer1():
        sb = s1_ref[pl.ds(j * BK, BK), :].astype(jnp.bfloat16)
        for idx, ar in enumerate(arefs):
            acc_ref[:, 32 * idx:32 * (idx + 1)] += jnp.dot(
                ar[...].astype(jnp.bfloat16), sb[:, 32 * idx:32 * (idx + 1)],
                preferred_element_type=jnp.float32)

    @pl.when(ph == 1)
    def _layer2():
        sb = s2_ref[pl.ds(j * BK, BK), :].astype(jnp.bfloat16)
        for idx, ar in enumerate(arefs):
            acc_ref[:, 16 * idx:16 * (idx + 1)] += jnp.dot(
                ar[...].astype(jnp.bfloat16), sb[:, 16 * idx:16 * (idx + 1)],
                preferred_element_type=jnp.float32)

    @pl.when((ph == 0) & (j == nk - 1))
    def _fin1():
        h = jnp.tanh(acc_ref[...] + b1_ref[0:1, :])
        s2_ref[pl.ds(i * BM, BM), :] = jnp.dot(
            h.astype(jnp.bfloat16), w2_ref[...].astype(jnp.bfloat16),
            preferred_element_type=jnp.float32)

    @pl.when((ph == 1) & (j == nk - 1))
    def _fin2():
        h2 = jnp.tanh(acc_ref[:, :64] + b2_ref[0:1, :])
        ob = jnp.dot(h2.astype(jnp.bfloat16), wl_ref[...].astype(jnp.bfloat16),
                     preferred_element_type=jnp.float32) + bl_ref[0:1, :]
        fused = jnp.dot(ob.astype(jnp.bfloat16), wg_ref[...].astype(jnp.bfloat16),
                        preferred_element_type=jnp.float32) + bf_ref[0:1, :]

        def lsm(z):
            m = jnp.max(z, axis=1, keepdims=True)
            e = z - m
            return e - jnp.log(jnp.sum(jnp.exp(e), axis=1, keepdims=True))

        lsm_f_ref[...] = lsm(fused)
        lsm_p1_ref[...] = lsm(ob[:, 8:16])
        lsm_p2_ref[...] = lsm(ob[:, 24:32])
        fused_ref[...] = fused


def kernel(x, A1, P1, A2, P2,
           W1_A1, b1_A1, W2_A1, b2_A1, Wl_A1, bl_A1,
           W1_A2, b1_A2, W2_A2, b2_A2, Wl_A2, bl_A2,
           W1_P1, b1_P1, W2_P1, b2_P1, Wl_P1, bl_P1,
           W1_P2, b1_P2, W2_P2, b2_P2, Wl_P2, bl_P2,
           Wf, bf):
    f32 = jnp.float32
    W1c = jnp.concatenate([W1_A1, W1_P1, W1_A2, W1_P2], axis=1)
    b1c = jnp.broadcast_to(
        jnp.concatenate([b1_A1, b1_P1, b1_A2, b1_P2])[None, :], (8, 128))
    W2bd = jax.scipy.linalg.block_diag(W2_A1, W2_P1, W2_A2, W2_P2)
    b2c = jnp.broadcast_to(
        jnp.concatenate([b2_A1, b2_P1, b2_A2, b2_P2])[None, :], (8, 64))
    Wlbd = jax.scipy.linalg.block_diag(Wl_A1, Wl_P1, Wl_A2, Wl_P2)
    blc = jnp.broadcast_to(
        jnp.concatenate([bl_A1, bl_P1, bl_A2, bl_P2])[None, :], (8, 32))
    Wg = jnp.zeros((32, 8), f32).at[0:8].set(Wf[0:8]).at[16:24].set(Wf[8:16])
    bfc = jnp.broadcast_to(bf[None, :], (8, 8))

    grid = (2, N // BM, N // BK)
    a_spec = pl.BlockSpec((BM, BK), lambda ph, i, j: (i, j))
    full = lambda r, c: pl.BlockSpec((r, c), lambda ph, i, j: (0, 0))
    o_spec = pl.BlockSpec((BM, 8), lambda ph, i, j: (i, 0))

    outs = pl.pallas_call(
        _mega_kernel,
        grid=grid,
        in_specs=[full(N, 128), a_spec, a_spec, a_spec, a_spec,
                  full(128, 128), full(8, 128), full(128, 64), full(8, 64),
                  full(64, 32), full(8, 32), full(32, 8), full(8, 8)],
        out_specs=[o_spec, o_spec, o_spec, o_spec],
        out_shape=[jax.ShapeDtypeStruct((N, 8), f32) for _ in range(4)],
        scratch_shapes=[pltpu.VMEM((N, 128), f32),
                        pltpu.VMEM((N, 64), f32),
                        pltpu.VMEM((BM, 128), f32)],
        compiler_params=pltpu.CompilerParams(
            dimension_semantics=("arbitrary", "arbitrary", "arbitrary")),
    )(x, A1, P1, A2, P2, W1c, b1c, W2bd, b2c, Wlbd, blc, Wg, bfc)

    return tuple(outs)


# speedup vs baseline: 1.0658x; 1.0240x over previous
import jax
import jax.numpy as jnp
from jax.experimental import pallas as pl
from jax.experimental.pallas import tpu as pltpu

N = 8192
BM = 256
BK = 4096


def _mega_kernel(x_ref, a1_ref, p1_ref, a2_ref, p2_ref,
                 w1_ref, b1_ref, w2_ref, b2_ref, wl_ref, bl_ref, wg_ref, bf_ref,
                 lsm_f_ref, lsm_p1_ref, lsm_p2_ref, fused_ref,
                 s1_ref, s2_ref, acc_ref):
    ph = pl.program_id(0)
    i = pl.program_id(1)
    j = pl.program_id(2)
    nk = pl.num_programs(2)

    @pl.when((ph == 0) & (i == 0))
    def _build_s1():
        xb = x_ref[pl.ds(j * BK, BK), :].astype(jnp.bfloat16)
        s1_ref[pl.ds(j * BK, BK), :] = jnp.dot(
            xb, w1_ref[...].astype(jnp.bfloat16),
            preferred_element_type=jnp.float32)

    @pl.when(j == 0)
    def _init():
        acc_ref[...] = jnp.zeros_like(acc_ref)

    arefs = (a1_ref, p1_ref, a2_ref, p2_ref)

    @pl.when(ph == 0)
    def _layer1():
        sb = s1_ref[pl.ds(j * BK, BK), :]
        for idx, ar in enumerate(arefs):
            acc_ref[:, 32 * idx:32 * (idx + 1)] += jnp.dot(
                ar[...], sb[:, 32 * idx:32 * (idx + 1)],
                precision=jax.lax.Precision.DEFAULT,
                preferred_element_type=jnp.float32)

    @pl.when(ph == 1)
    def _layer2():
        sb = s2_ref[pl.ds(j * BK, BK), :]
        for idx, ar in enumerate(arefs):
            acc_ref[:, 16 * idx:16 * (idx + 1)] += jnp.dot(
                ar[...], sb[:, 16 * idx:16 * (idx + 1)],
                precision=jax.lax.Precision.DEFAULT,
                preferred_element_type=jnp.float32)

    @pl.when((ph == 0) & (j == nk - 1))
    def _fin1():
        h = jnp.tanh(acc_ref[...] + b1_ref[0:1, :])
        s2_ref[pl.ds(i * BM, BM), :] = jnp.dot(
            h.astype(jnp.bfloat16), w2_ref[...].astype(jnp.bfloat16),
            preferred_element_type=jnp.float32)

    @pl.when((ph == 1) & (j == nk - 1))
    def _fin2():
        h2 = jnp.tanh(acc_ref[:, :64] + b2_ref[0:1, :])
        ob = jnp.dot(h2.astype(jnp.bfloat16), wl_ref[...].astype(jnp.bfloat16),
                     preferred_element_type=jnp.float32) + bl_ref[0:1, :]
        fused = jnp.dot(ob.astype(jnp.bfloat16), wg_ref[...].astype(jnp.bfloat16),
                        preferred_element_type=jnp.float32) + bf_ref[0:1, :]

        def lsm(z):
            m = jnp.max(z, axis=1, keepdims=True)
            e = z - m
            return e - jnp.log(jnp.sum(jnp.exp(e), axis=1, keepdims=True))

        lsm_f_ref[...] = lsm(fused)
        lsm_p1_ref[...] = lsm(ob[:, 8:16])
        lsm_p2_ref[...] = lsm(ob[:, 24:32])
        fused_ref[...] = fused


def kernel(x, A1, P1, A2, P2,
           W1_A1, b1_A1, W2_A1, b2_A1, Wl_A1, bl_A1,
           W1_A2, b1_A2, W2_A2, b2_A2, Wl_A2, bl_A2,
           W1_P1, b1_P1, W2_P1, b2_P1, Wl_P1, bl_P1,
           W1_P2, b1_P2, W2_P2, b2_P2, Wl_P2, bl_P2,
           Wf, bf):
    f32 = jnp.float32
    W1c = jnp.concatenate([W1_A1, W1_P1, W1_A2, W1_P2], axis=1)
    b1c = jnp.broadcast_to(
        jnp.concatenate([b1_A1, b1_P1, b1_A2, b1_P2])[None, :], (8, 128))
    W2bd = jax.scipy.linalg.block_diag(W2_A1, W2_P1, W2_A2, W2_P2)
    b2c = jnp.broadcast_to(
        jnp.concatenate([b2_A1, b2_P1, b2_A2, b2_P2])[None, :], (8, 64))
    Wlbd = jax.scipy.linalg.block_diag(Wl_A1, Wl_P1, Wl_A2, Wl_P2)
    blc = jnp.broadcast_to(
        jnp.concatenate([bl_A1, bl_P1, bl_A2, bl_P2])[None, :], (8, 32))
    Wg = jnp.zeros((32, 8), f32).at[0:8].set(Wf[0:8]).at[16:24].set(Wf[8:16])
    bfc = jnp.broadcast_to(bf[None, :], (8, 8))

    grid = (2, N // BM, N // BK)
    a_spec = pl.BlockSpec((BM, BK), lambda ph, i, j: (i, j))
    full = lambda r, c: pl.BlockSpec((r, c), lambda ph, i, j: (0, 0))
    o_spec = pl.BlockSpec((BM, 8), lambda ph, i, j: (i, 0))

    outs = pl.pallas_call(
        _mega_kernel,
        grid=grid,
        in_specs=[full(N, 128), a_spec, a_spec, a_spec, a_spec,
                  full(128, 128), full(8, 128), full(128, 64), full(8, 64),
                  full(64, 32), full(8, 32), full(32, 8), full(8, 8)],
        out_specs=[o_spec, o_spec, o_spec, o_spec],
        out_shape=[jax.ShapeDtypeStruct((N, 8), f32) for _ in range(4)],
        scratch_shapes=[pltpu.VMEM((N, 128), f32),
                        pltpu.VMEM((N, 64), f32),
                        pltpu.VMEM((BM, 128), f32)],
        compiler_params=pltpu.CompilerParams(
            dimension_semantics=("arbitrary", "arbitrary", "arbitrary")),
    )(x, A1, P1, A2, P2, W1c, b1c, W2bd, b2c, Wlbd, blc, Wg, bfc)

    return tuple(outs)
